# baseline (speedup 1.0000x reference)
import functools
import jax
import jax.numpy as jnp
from jax import lax
from jax.experimental import pallas as pl
from jax.experimental.pallas import tpu as pltpu, tpu_sc as plsc

POOL = 7
PP = POOL * POOL
GP = 56


def _make_sc_kernel(NB, C, Hs, B, boxes_per_w):
    NLVL = len(Hs)
    NV = C // 16
    assert boxes_per_w % 2 == 1

    mesh = plsc.VectorSubcoreMesh(core_axis_name="c", subcore_axis_name="s")
    info = plsc.get_sparse_core_info()
    NC = info.num_cores

    @functools.partial(
        pl.kernel,
        mesh=mesh,
        out_type=jax.ShapeDtypeStruct((NB, PP, C), jnp.float32),
        scratch_types=[
            pltpu.VMEM((boxes_per_w * 8 + 16,), jnp.float32),
            pltpu.VMEM((16,), jnp.int32),
            pltpu.VMEM((2, GP), jnp.int32),
            pltpu.VMEM((2, GP), jnp.int32),
            pltpu.VMEM((4, 64), jnp.float32),
            pltpu.VMEM((4, 64), jnp.float32),
            pltpu.VMEM((2, GP, C // 2), jnp.int32),
            pltpu.VMEM((2, GP, C // 2), jnp.int32),
            pltpu.VMEM((PP, C), jnp.float32),
            pltpu.SemaphoreType.DMA,
            pltpu.SemaphoreType.DMA,
            pltpu.SemaphoreType.DMA,
        ],
    )
    def k(t2_hbm, t3_hbm, t4_hbm, t5_hbm, boxes_hbm, img_hbm, out_hbm,
          bx_v, img_v, idx_a, idx_b, w_e, w_o, rows_a, rows_b, out_v,
          gsem_a, gsem_b, osem):
        tbls = (t2_hbm, t3_hbm, t4_hbm, t5_hbm)
        wid = lax.axis_index("s") * NC + lax.axis_index("c")
        box0 = wid * boxes_per_w

        pltpu.async_copy(img_hbm, img_v, gsem_a).wait()
        pltpu.async_copy(
            boxes_hbm.at[pl.ds(box0 * 8, boxes_per_w * 8)],
            bx_v.at[pl.ds(0, boxes_per_w * 8)],
            gsem_a,
        ).wait()

        imgv = img_v[pl.ds(0, 16)]
        area = (imgv[0] * imgv[1]).astype(jnp.float32)
        c3 = 224.0 * 224.0 * 2.0 ** -3
        c4 = 224.0 * 224.0 * 2.0 ** -1
        c5 = 224.0 * 224.0 * 2.0 ** 1

        def issue_phase(j, ph, idx_v, w_v, rows_v, gsem):
            bv = bx_v[pl.ds(j * 8, 16)]
            y1 = bv[0]
            x1 = bv[1]
            y2 = bv[2]
            x2 = bv[3]
            Hb = y2 - y1
            Wb = x2 - x1
            va = Hb * Wb * area
            lvl = (
                2
                + (va >= c3).astype(jnp.int32)
                + (va >= c4).astype(jnp.int32)
                + (va >= c5).astype(jnp.int32)
            )
            Hm = jnp.int32(Hs[0])
            for li in range(1, NLVL):
                sel = lvl >= (2 + li)
                Hm = jnp.where(sel, jnp.int32(Hs[li]), Hm)
            b = bv[4].astype(jnp.int32)
            base = b * Hm * Hm
            hm1 = Hm - 1
            hm1f = hm1.astype(jnp.float32)
            ay = y1 * hm1f
            dy = Hb * hm1f * (1.0 / (POOL - 1))
            ax = x1 * hm1f
            dx = Wb * hm1f * (1.0 / (POOL - 1))

            lanes = lax.iota(jnp.int32, 16)
            magic = (1 << 16) // POOL + 1
            for o in (0, 16, 32, 40):
                p = lanes + o
                iy = lax.shift_right_logical(p * magic, 16)
                ix = p - iy * POOL
                in_y = ay + iy.astype(jnp.float32) * dy
                in_x = ax + ix.astype(jnp.float32) * dx
                ylo = in_y.astype(jnp.int32)
                xlo = in_x.astype(jnp.int32)
                yl = in_y - ylo.astype(jnp.float32)
                xl = in_x - xlo.astype(jnp.float32)
                ylo = jnp.minimum(ylo, hm1)
                xlo = jnp.minimum(xlo, hm1)
                xhi = jnp.minimum(xlo + 1, hm1)
                if ph == 0:
                    yr = ylo
                    wy = 1.0 - yl
                else:
                    yr = jnp.minimum(ylo + 1, hm1)
                    wy = yl
                r = base + yr * Hm
                idx_v[0, pl.ds(o, 16)] = r + xlo
                idx_v[1, pl.ds(o, 16)] = r + xhi
                w_v[2 * ph + 0, pl.ds(o, 16)] = wy * (1.0 - xl)
                w_v[2 * ph + 1, pl.ds(o, 16)] = wy * xl

            for li in range(NLVL):
                @pl.when(lvl == 2 + li)
                def _():
                    for c in range(2):
                        pltpu.async_copy(
                            tbls[li].at[idx_v.at[c]], rows_v.at[c], gsem
                        )

        def finish_phase(ph, idx_v, w_v, rows_v, gsem):
            for c in range(2):
                pltpu.make_async_copy(
                    t2_hbm.at[idx_v.at[c]], rows_v.at[c], gsem
                ).wait()

            hi_mask = jnp.int32(-65536)

            def bf2f32(v):
                lo = lax.bitcast_convert_type(lax.shift_left(v, 16), jnp.float32)
                hi = lax.bitcast_convert_type(lax.bitwise_and(v, hi_mask), jnp.float32)
                return lo, hi

            def per_sample(g, _):
                w0 = w_v[2 * ph + 0, pl.ds(g, 16)][0]
                w1 = w_v[2 * ph + 1, pl.ds(g, 16)][0]
                for q in range(C // 32):
                    s = pl.ds(q * 16, 16)
                    lo0, hi0 = bf2f32(rows_v[0, g, s])
                    lo1, hi1 = bf2f32(rows_v[1, g, s])
                    sa = pl.ds(q * 32, 16)
                    sb = pl.ds(q * 32 + 16, 16)
                    acc_a = w0 * lo0 + w1 * lo1
                    acc_b = w0 * hi0 + w1 * hi1
                    if ph == 0:
                        out_v[g, sa] = acc_a
                        out_v[g, sb] = acc_b
                    else:
                        out_v[g, sa] = out_v[g, sa] + acc_a
                        out_v[g, sb] = out_v[g, sb] + acc_b
                return 0

            lax.fori_loop(0, PP, per_sample, 0)

        def write_out(j):
            @pl.when(box0 + j < NB)
            def _():
                pltpu.async_copy(out_v, out_hbm.at[box0 + j], osem).wait()

        issue_phase(0, 0, idx_a, w_e, rows_a, gsem_a)

        def pair(t, _):
            j = 2 * t
            issue_phase(j, 1, idx_b, w_e, rows_b, gsem_b)
            finish_phase(0, idx_a, w_e, rows_a, gsem_a)
            issue_phase(j + 1, 0, idx_a, w_o, rows_a, gsem_a)
            finish_phase(1, idx_b, w_e, rows_b, gsem_b)
            write_out(j)
            issue_phase(j + 1, 1, idx_b, w_o, rows_b, gsem_b)
            finish_phase(0, idx_a, w_o, rows_a, gsem_a)
            issue_phase(j + 2, 0, idx_a, w_e, rows_a, gsem_a)
            finish_phase(1, idx_b, w_o, rows_b, gsem_b)
            write_out(j + 1)
            return 0

        lax.fori_loop(0, (boxes_per_w - 1) // 2, pair, 0)
        j_last = boxes_per_w - 1
        issue_phase(j_last, 1, idx_b, w_e, rows_b, gsem_b)
        finish_phase(0, idx_a, w_e, rows_a, gsem_a)
        finish_phase(1, idx_b, w_e, rows_b, gsem_b)
        write_out(j_last)

    return k


def kernel(boxes, image_shape, P2, P3, P4, P5):
    B, N = boxes.shape[0], boxes.shape[1]
    C = P2.shape[-1]
    maps = [P2, P3, P4, P5]
    Hs = tuple(m.shape[1] for m in maps)

    NW = 32
    boxes_per_w = -(-(B * N) // NW)
    if boxes_per_w % 2 == 0:
        boxes_per_w += 1
    NB_PAD = boxes_per_w * NW
    fb = boxes.reshape(-1, 4)
    pad = NB_PAD - B * N
    fb = jnp.pad(fb, ((0, pad), (0, 0)), constant_values=0.25)
    bidx = jnp.pad(
        jnp.repeat(jnp.arange(B, dtype=jnp.float32), N), (0, pad)
    )[:, None]
    fb8 = jnp.concatenate(
        [fb, bidx, jnp.zeros((NB_PAD, 3), jnp.float32)], axis=1
    ).reshape(-1)
    img16 = jnp.pad(image_shape, (0, 14))

    k = _make_sc_kernel(B * N, C, Hs, B, boxes_per_w)
    tbls = [
        jax.lax.bitcast_convert_type(
            m.reshape(-1, C // 32, 2, 16)
            .swapaxes(2, 3)
            .astype(jnp.bfloat16)
            .reshape(-1, C // 2, 2),
            jnp.int32,
        )
        for m in maps
    ]
    out = k(*tbls, fb8, img16)
    return out.reshape(B, N, POOL, POOL, C)


# speedup vs baseline: 1.6837x; 1.6837x over previous
import functools
import jax
import jax.numpy as jnp
from jax import lax
from jax.experimental import pallas as pl
from jax.experimental.pallas import tpu as pltpu, tpu_sc as plsc

POOL = 7
PP = POOL * POOL
GP = 56


def _make_sc_kernel(NB, C, Hs, B, boxes_per_w):
    NLVL = len(Hs)
    NV = C // 16
    assert boxes_per_w % 2 == 1

    mesh = plsc.VectorSubcoreMesh(core_axis_name="c", subcore_axis_name="s")
    info = plsc.get_sparse_core_info()
    NC = info.num_cores

    @functools.partial(
        pl.kernel,
        mesh=mesh,
        out_type=jax.ShapeDtypeStruct((NB, PP, C), jnp.float32),
        scratch_types=[
            pltpu.VMEM((boxes_per_w * 8 + 16,), jnp.float32),
            pltpu.VMEM((16,), jnp.int32),
            pltpu.VMEM((2, GP), jnp.int32),
            pltpu.VMEM((2, GP), jnp.int32),
            pltpu.VMEM((4, 64), jnp.float32),
            pltpu.VMEM((4, 64), jnp.float32),
            pltpu.VMEM((2, GP, C), jnp.float32),
            pltpu.VMEM((2, GP, C), jnp.float32),
            pltpu.VMEM((PP, C), jnp.float32),
            pltpu.SemaphoreType.DMA,
            pltpu.SemaphoreType.DMA,
            pltpu.SemaphoreType.DMA,
        ],
    )
    def k(t2_hbm, t3_hbm, t4_hbm, t5_hbm, boxes_hbm, img_hbm, out_hbm,
          bx_v, img_v, idx_a, idx_b, w_e, w_o, rows_a, rows_b, out_v,
          gsem_a, gsem_b, osem):
        tbls = (t2_hbm, t3_hbm, t4_hbm, t5_hbm)
        wid = lax.axis_index("s") * NC + lax.axis_index("c")
        box0 = wid * boxes_per_w

        pltpu.async_copy(img_hbm, img_v, gsem_a).wait()
        pltpu.async_copy(
            boxes_hbm.at[pl.ds(box0 * 8, boxes_per_w * 8)],
            bx_v.at[pl.ds(0, boxes_per_w * 8)],
            gsem_a,
        ).wait()

        imgv = img_v[pl.ds(0, 16)]
        area = (imgv[0] * imgv[1]).astype(jnp.float32)
        c3 = 224.0 * 224.0 * 2.0 ** -3
        c4 = 224.0 * 224.0 * 2.0 ** -1
        c5 = 224.0 * 224.0 * 2.0 ** 1

        def issue_phase(j, ph, idx_v, w_v, rows_v, gsem):
            bv = bx_v[pl.ds(j * 8, 16)]
            y1 = bv[0]
            x1 = bv[1]
            y2 = bv[2]
            x2 = bv[3]
            Hb = y2 - y1
            Wb = x2 - x1
            va = Hb * Wb * area
            lvl = (
                2
                + (va >= c3).astype(jnp.int32)
                + (va >= c4).astype(jnp.int32)
                + (va >= c5).astype(jnp.int32)
            )
            Hm = jnp.int32(Hs[0])
            for li in range(1, NLVL):
                sel = lvl >= (2 + li)
                Hm = jnp.where(sel, jnp.int32(Hs[li]), Hm)
            b = bv[4].astype(jnp.int32)
            base = b * Hm * Hm
            hm1 = Hm - 1
            hm1f = hm1.astype(jnp.float32)
            ay = y1 * hm1f
            dy = Hb * hm1f * (1.0 / (POOL - 1))
            ax = x1 * hm1f
            dx = Wb * hm1f * (1.0 / (POOL - 1))

            lanes = lax.iota(jnp.int32, 16)
            magic = (1 << 16) // POOL + 1
            for o in (0, 16, 32, 40):
                p = lanes + o
                iy = lax.shift_right_logical(p * magic, 16)
                ix = p - iy * POOL
                in_y = ay + iy.astype(jnp.float32) * dy
                in_x = ax + ix.astype(jnp.float32) * dx
                ylo = in_y.astype(jnp.int32)
                xlo = in_x.astype(jnp.int32)
                yl = in_y - ylo.astype(jnp.float32)
                xl = in_x - xlo.astype(jnp.float32)
                ylo = jnp.minimum(ylo, hm1)
                xlo = jnp.minimum(xlo, hm1)
                xhi = jnp.minimum(xlo + 1, hm1)
                if ph == 0:
                    yr = ylo
                    wy = 1.0 - yl
                else:
                    yr = jnp.minimum(ylo + 1, hm1)
                    wy = yl
                r = base + yr * Hm
                idx_v[0, pl.ds(o, 16)] = r + xlo
                idx_v[1, pl.ds(o, 16)] = r + xhi
                w_v[2 * ph + 0, pl.ds(o, 16)] = wy * (1.0 - xl)
                w_v[2 * ph + 1, pl.ds(o, 16)] = wy * xl

            for li in range(NLVL):
                @pl.when(lvl == 2 + li)
                def _():
                    for c in range(2):
                        pltpu.async_copy(
                            tbls[li].at[idx_v.at[c]], rows_v.at[c], gsem
                        )

        def finish_phase(ph, idx_v, w_v, rows_v, gsem):
            for c in range(2):
                pltpu.make_async_copy(
                    t2_hbm.at[idx_v.at[c]], rows_v.at[c], gsem
                ).wait()

            def per_sample(g, _):
                w0 = w_v[2 * ph + 0, pl.ds(g, 16)][0]
                w1 = w_v[2 * ph + 1, pl.ds(g, 16)][0]
                for t in range(NV):
                    s = pl.ds(t * 16, 16)
                    acc = w0 * rows_v[0, g, s] + w1 * rows_v[1, g, s]
                    if ph == 0:
                        out_v[g, s] = acc
                    else:
                        out_v[g, s] = out_v[g, s] + acc
                return 0

            lax.fori_loop(0, PP, per_sample, 0)

        def write_out(j):
            @pl.when(box0 + j < NB)
            def _():
                pltpu.async_copy(out_v, out_hbm.at[box0 + j], osem).wait()

        issue_phase(0, 0, idx_a, w_e, rows_a, gsem_a)

        def pair(t, _):
            j = 2 * t
            issue_phase(j, 1, idx_b, w_e, rows_b, gsem_b)
            finish_phase(0, idx_a, w_e, rows_a, gsem_a)
            issue_phase(j + 1, 0, idx_a, w_o, rows_a, gsem_a)
            finish_phase(1, idx_b, w_e, rows_b, gsem_b)
            write_out(j)
            issue_phase(j + 1, 1, idx_b, w_o, rows_b, gsem_b)
            finish_phase(0, idx_a, w_o, rows_a, gsem_a)
            issue_phase(j + 2, 0, idx_a, w_e, rows_a, gsem_a)
            finish_phase(1, idx_b, w_o, rows_b, gsem_b)
            write_out(j + 1)
            return 0

        lax.fori_loop(0, (boxes_per_w - 1) // 2, pair, 0)
        j_last = boxes_per_w - 1
        issue_phase(j_last, 1, idx_b, w_e, rows_b, gsem_b)
        finish_phase(0, idx_a, w_e, rows_a, gsem_a)
        finish_phase(1, idx_b, w_e, rows_b, gsem_b)
        write_out(j_last)

    return k


def kernel(boxes, image_shape, P2, P3, P4, P5):
    B, N = boxes.shape[0], boxes.shape[1]
    C = P2.shape[-1]
    maps = [P2, P3, P4, P5]
    Hs = tuple(m.shape[1] for m in maps)

    NW = 32
    boxes_per_w = -(-(B * N) // NW)
    if boxes_per_w % 2 == 0:
        boxes_per_w += 1
    NB_PAD = boxes_per_w * NW
    fb = boxes.reshape(-1, 4)
    pad = NB_PAD - B * N
    fb = jnp.pad(fb, ((0, pad), (0, 0)), constant_values=0.25)
    bidx = jnp.pad(
        jnp.repeat(jnp.arange(B, dtype=jnp.float32), N), (0, pad)
    )[:, None]
    fb8 = jnp.concatenate(
        [fb, bidx, jnp.zeros((NB_PAD, 3), jnp.float32)], axis=1
    ).reshape(-1)
    img16 = jnp.pad(image_shape, (0, 14))

    k = _make_sc_kernel(B * N, C, Hs, B, boxes_per_w)
    out = k(*[m.reshape(-1, C) for m in maps], fb8, img16)
    return out.reshape(B, N, POOL, POOL, C)
